# Initial kernel scaffold; baseline (speedup 1.0000x reference)
import jax
import jax.numpy as jnp
from jax.experimental import pallas as pl


def kernel(support_x, support_edge_index, support_batch, query_x, query_edge_index, query_batch, W1, b1, W2, b2, Wp, bp, Wc, bc):
    raise NotImplementedError("write your pallas kernel here")


# speedup vs baseline: 7.1361x; 7.1361x over previous
import functools

import jax
import jax.numpy as jnp
from jax import lax
from jax.experimental import pallas as pl
from jax.experimental.pallas import tpu as pltpu
from jax.experimental.pallas import tpu_sc as plsc

N = 10000
E = 320000
D = 128
H = 128
OUT = 128
G = 256

NC = 2
NS = 16
NW = NC * NS

CH = 128
EPT = E // NW
NFULL = EPT // CH
TAIL = EPT - NFULL * CH
RPT = 624
CZ = 104


def _sc_scatter_add(x, src, dst):
    mesh = plsc.VectorSubcoreMesh(core_axis_name="c", subcore_axis_name="s")

    @functools.partial(
        pl.kernel,
        mesh=mesh,
        out_type=jax.ShapeDtypeStruct((NC, N, D), jnp.float32),
        scratch_types=[
            pltpu.VMEM((CH,), jnp.int32),
            pltpu.VMEM((CH,), jnp.int32),
            pltpu.VMEM((CH, D), jnp.float32),
            pltpu.VMEM((TAIL,), jnp.int32),
            pltpu.VMEM((TAIL,), jnp.int32),
            pltpu.VMEM((TAIL, D), jnp.float32),
            pltpu.VMEM((CZ, D), jnp.float32),
            pltpu.VMEM_SHARED((N, D), jnp.float32),
            pltpu.SemaphoreType.DMA,
        ],
    )
    def body(x_hbm, src_hbm, dst_hbm, out_hbm,
             sbuf, dbuf, rows, sbuf_t, dbuf_t, rows_t, zbuf, acc, sem):
        cid = lax.axis_index("c")
        sid = lax.axis_index("s")

        z16 = jnp.zeros((16,), jnp.float32)

        def zrow(r, carry):
            for j in range(D // 16):
                zbuf[r, pl.ds(j * 16, 16)] = z16
            return carry

        lax.fori_loop(0, CZ, zrow, 0)
        r0 = sid * RPT
        for k in range(RPT // CZ):
            pltpu.sync_copy(zbuf, acc.at[pl.ds(r0 + k * CZ, CZ)])

        @pl.when(sid == NS - 1)
        def _():
            pltpu.sync_copy(zbuf.at[pl.ds(0, 16)], acc.at[pl.ds(NS * RPT, 16)])

        plsc.subcore_barrier()

        wid = cid * NS + sid
        base = wid * EPT

        def step(j, carry):
            off = base + j * CH
            pltpu.sync_copy(src_hbm.at[pl.ds(off, CH)], sbuf)
            pltpu.sync_copy(dst_hbm.at[pl.ds(off, CH)], dbuf)
            pltpu.async_copy(x_hbm.at[sbuf], rows, sem).wait()
            pltpu.sync_copy(rows, acc.at[dbuf], add=True)
            return carry

        lax.fori_loop(0, NFULL, step, 0)

        offt = base + NFULL * CH
        pltpu.sync_copy(src_hbm.at[pl.ds(offt, TAIL)], sbuf_t)
        pltpu.sync_copy(dst_hbm.at[pl.ds(offt, TAIL)], dbuf_t)
        pltpu.async_copy(x_hbm.at[sbuf_t], rows_t, sem).wait()
        pltpu.sync_copy(rows_t, acc.at[dbuf_t], add=True)

        plsc.subcore_barrier()

        for k in range(RPT // CZ):
            rr = r0 + k * CZ
            pltpu.sync_copy(acc.at[pl.ds(rr, CZ)], out_hbm.at[cid, pl.ds(rr, CZ)])

        @pl.when(sid == NS - 1)
        def _():
            pltpu.sync_copy(acc.at[pl.ds(NS * RPT, 16)],
                            out_hbm.at[cid, pl.ds(NS * RPT, 16)])

    return body(x, src, dst)


BLK = 2000


def _tc_head(x, agg, batch2d, W1, b1, W2, b2, Wp, bp, Wc, bc):
    grid = N // BLK

    def body(x_ref, agg_ref, b_ref, W1_ref, b1_ref, W2_ref, b2_ref,
             Wp_ref, bp_ref, Wc_ref, bc_ref, out_ref, sums, cnt):
        i = pl.program_id(0)
        t = x_ref[...] + agg_ref[0] + agg_ref[1]
        h = jnp.dot(t, W1_ref[...], preferred_element_type=jnp.float32) + b1_ref[...]
        h = jnp.maximum(h, 0.0)
        h2 = jnp.dot(h, W2_ref[...], preferred_element_type=jnp.float32) + b2_ref[...]
        seg = b_ref[...].reshape(1, BLK)
        gids = lax.broadcasted_iota(jnp.int32, (G, BLK), 0)
        m = (gids == seg).astype(jnp.float32)
        psum = jnp.dot(m, h2, preferred_element_type=jnp.float32)
        pcnt = jnp.sum(m, axis=1, keepdims=True)

        @pl.when(i == 0)
        def _():
            sums[...] = psum
            cnt[...] = pcnt

        @pl.when(i > 0)
        def _():
            sums[...] += psum
            cnt[...] += pcnt

        @pl.when(i == grid - 1)
        def _():
            pooled = sums[...] / jnp.maximum(cnt[...], 1.0)
            emb = jnp.dot(pooled, Wp_ref[...], preferred_element_type=jnp.float32) + bp_ref[...]
            out_ref[...] = jnp.dot(emb, Wc_ref[...], preferred_element_type=jnp.float32) + bc_ref[...]

    return pl.pallas_call(
        body,
        grid=(grid,),
        in_specs=[
            pl.BlockSpec((BLK, D), lambda i: (i, 0)),
            pl.BlockSpec((NC, BLK, D), lambda i: (0, i, 0)),
            pl.BlockSpec((BLK, 1), lambda i: (i, 0)),
            pl.BlockSpec((D, H), lambda i: (0, 0)),
            pl.BlockSpec((1, H), lambda i: (0, 0)),
            pl.BlockSpec((H, H), lambda i: (0, 0)),
            pl.BlockSpec((1, H), lambda i: (0, 0)),
            pl.BlockSpec((H, OUT), lambda i: (0, 0)),
            pl.BlockSpec((1, OUT), lambda i: (0, 0)),
            pl.BlockSpec((OUT, 1), lambda i: (0, 0)),
            pl.BlockSpec((1, 1), lambda i: (0, 0)),
        ],
        out_specs=pl.BlockSpec((G, 1), lambda i: (0, 0)),
        out_shape=jax.ShapeDtypeStruct((G, 1), jnp.float32),
        scratch_shapes=[
            pltpu.VMEM((G, H), jnp.float32),
            pltpu.VMEM((G, 1), jnp.float32),
        ],
    )(x, agg, batch2d, W1, b1, W2, b2, Wp, bp, Wc, bc)


def kernel(support_x, support_edge_index, support_batch,
           query_x, query_edge_index, query_batch,
           W1, b1, W2, b2, Wp, bp, Wc, bc):
    src = query_edge_index[0]
    dst = query_edge_index[1]
    agg = _sc_scatter_add(query_x, src, dst)
    return _tc_head(
        query_x, agg, query_batch.reshape(N, 1),
        W1, b1.reshape(1, H), W2, b2.reshape(1, H),
        Wp, bp.reshape(1, OUT), Wc, bc.reshape(1, 1))
